# Initial kernel scaffold; baseline (speedup 1.0000x reference)
import jax
import jax.numpy as jnp
from jax.experimental import pallas as pl


def kernel(discrete_actions, discrete_embed_weight):
    raise NotImplementedError("write your pallas kernel here")


# speedup vs baseline: 6.5386x; 6.5386x over previous
import functools

import jax
import jax.numpy as jnp
from jax import lax
from jax.experimental import pallas as pl
from jax.experimental.pallas import tpu as pltpu
from jax.experimental.pallas import tpu_sc as plsc

_CHUNK = 128
_NBUF = 4


def _embed_lookup(table, idx2d, n_rows, n_workers):
    rows_per_w = n_rows // n_workers
    chunks_per_w = rows_per_w // _CHUNK
    d = table.shape[1]
    mesh = plsc.VectorSubcoreMesh(core_axis_name="c", subcore_axis_name="s")
    num_cores = mesh.num_cores

    @functools.partial(
        pl.kernel,
        out_type=jax.ShapeDtypeStruct((n_rows, d), table.dtype),
        mesh=mesh,
        scratch_types=[
            pltpu.VMEM((chunks_per_w, _CHUNK), jnp.int32),
            pltpu.VMEM((_NBUF, _CHUNK, d), table.dtype),
            [pltpu.SemaphoreType.DMA] * _NBUF,
        ],
    )
    def run(table_hbm, idx_hbm, out_hbm, idx_v, rows_v, sems):
        wid = lax.axis_index("s") * num_cores + lax.axis_index("c")
        base = wid * rows_per_w

        pltpu.sync_copy(idx_hbm.at[pl.ds(wid * chunks_per_w, chunks_per_w)],
                        idx_v)

        def gather(chunk, buf):
            return pltpu.make_async_copy(
                table_hbm.at[idx_v.at[chunk]], rows_v.at[buf], sems[buf])

        for b in range(_NBUF):
            gather(b, b).start()

        def outer(t, carry):
            for b in range(_NBUF):
                c = t * _NBUF + b
                gather(c, b).wait()
                pltpu.sync_copy(rows_v.at[b],
                                out_hbm.at[pl.ds(base + c * _CHUNK, _CHUNK)])
                gather(c + _NBUF, b).start()
            return carry

        lax.fori_loop(0, chunks_per_w // _NBUF - 1, outer, 0)

        for b in range(_NBUF):
            c = chunks_per_w - _NBUF + b
            gather(c, b).wait()
            pltpu.sync_copy(rows_v.at[b],
                            out_hbm.at[pl.ds(base + c * _CHUNK, _CHUNK)])

    return run(table, idx2d)


def kernel(discrete_actions, discrete_embed_weight):
    bsz, seq = discrete_actions.shape
    n_rows = bsz * seq
    idx2d = discrete_actions.astype(jnp.int32).reshape(n_rows // _CHUNK, _CHUNK)
    out = _embed_lookup(discrete_embed_weight, idx2d, n_rows, n_workers=32)
    return out.reshape(bsz, seq, discrete_embed_weight.shape[1])


# speedup vs baseline: 16.0426x; 2.4535x over previous
import functools

import jax
import jax.numpy as jnp
from jax import lax
from jax.experimental import pallas as pl
from jax.experimental.pallas import tpu as pltpu
from jax.experimental.pallas import tpu_sc as plsc

_CHUNK = 128
_NBUF = 4


def _embed_lookup(table, idx2d, n_rows, n_workers):
    rows_per_w = n_rows // n_workers
    chunks_per_w = rows_per_w // _CHUNK
    d = table.shape[1]
    mesh = plsc.VectorSubcoreMesh(core_axis_name="c", subcore_axis_name="s")
    num_cores = mesh.num_cores

    @functools.partial(
        pl.kernel,
        out_type=jax.ShapeDtypeStruct((n_rows, d), table.dtype),
        mesh=mesh,
        scratch_types=[
            pltpu.VMEM((chunks_per_w, _CHUNK), jnp.int32),
            pltpu.VMEM((_NBUF, _CHUNK, d), table.dtype),
            pltpu.VMEM_SHARED(table.shape, table.dtype),
            [pltpu.SemaphoreType.DMA] * _NBUF,
        ],
    )
    def run(table_hbm, idx_hbm, out_hbm, idx_v, rows_v, table_sp, sems):
        sid = lax.axis_index("s")
        wid = sid * num_cores + lax.axis_index("c")
        base = wid * rows_per_w

        @pl.when(sid == 0)
        def _():
            pltpu.sync_copy(table_hbm, table_sp)

        pltpu.sync_copy(idx_hbm.at[pl.ds(wid * chunks_per_w, chunks_per_w)],
                        idx_v)
        plsc.subcore_barrier()

        def gather(chunk, buf):
            return pltpu.make_async_copy(
                table_sp.at[idx_v.at[chunk]], rows_v.at[buf], sems[buf])

        for b in range(_NBUF):
            gather(b, b).start()

        def outer(t, carry):
            for b in range(_NBUF):
                c = t * _NBUF + b
                gather(c, b).wait()
                pltpu.sync_copy(rows_v.at[b],
                                out_hbm.at[pl.ds(base + c * _CHUNK, _CHUNK)])
                gather(c + _NBUF, b).start()
            return carry

        lax.fori_loop(0, chunks_per_w // _NBUF - 1, outer, 0)

        for b in range(_NBUF):
            c = chunks_per_w - _NBUF + b
            gather(c, b).wait()
            pltpu.sync_copy(rows_v.at[b],
                            out_hbm.at[pl.ds(base + c * _CHUNK, _CHUNK)])

    return run(table, idx2d)


def kernel(discrete_actions, discrete_embed_weight):
    bsz, seq = discrete_actions.shape
    n_rows = bsz * seq
    idx2d = discrete_actions.astype(jnp.int32).reshape(n_rows // _CHUNK, _CHUNK)
    out = _embed_lookup(discrete_embed_weight, idx2d, n_rows, n_workers=32)
    return out.reshape(bsz, seq, discrete_embed_weight.shape[1])
